# baseline (speedup 1.0000x reference)
import functools
import math

import jax
import jax.numpy as jnp
from jax import lax
from jax.experimental import pallas as pl
from jax.experimental.pallas import tpu as pltpu
from jax.experimental.pallas import tpu_sc as plsc

_D = 64
_LANES = 16
_NC = 2
_NS = 16
_NW = _NC * _NS
_K = 4
_SCALE = math.sqrt(_D)
_G0 = 104
_G1 = 96


def _body(x_ref, table_ref, out_ref, idx_v, gbuf, sbuf, gsems, osems):
    wid = lax.axis_index("s") * _NC + lax.axis_index("c")
    rows_per_w = x_ref.shape[0] // _NW
    rbase = wid * rows_per_w
    n_groups = 2 * rows_per_w

    pltpu.sync_copy(x_ref.at[pl.ds(rbase, rows_per_w)], idx_v)

    def grp(g, b):
        row = g // 2
        col = (_G0, 0) if b % 2 == 0 else (_G1, _G0)
        return row, col[1], col[0]

    def start_gather(b, g):
        row, col, w = grp(g, b)
        pltpu.async_copy(
            table_ref.at[idx_v.at[row, pl.ds(col, w)]],
            gbuf.at[b, pl.ds(0, w)],
            gsems.at[b],
        )

    def wait_gather(b, g):
        row, col, w = grp(g, b)
        pltpu.make_async_copy(
            table_ref.at[idx_v.at[row, pl.ds(col, w)]],
            gbuf.at[b, pl.ds(0, w)],
            gsems.at[b],
        ).wait()

    def start_out(b, g):
        row, col, w = grp(g, b)
        pltpu.async_copy(
            sbuf.at[b, pl.ds(0, w)],
            out_ref.at[rbase + row, pl.ds(col, w)],
            osems.at[b],
        )

    def wait_out(b):
        w = _G0 if b % 2 == 0 else _G1
        pltpu.make_async_copy(
            sbuf.at[b, pl.ds(0, w)],
            out_ref.at[0, pl.ds(0, w)],
            osems.at[b],
        ).wait()

    for b in range(_K):
        start_gather(b, b)

    @pl.loop(0, n_groups // _K)
    def _rounds(r):
        for b in range(_K):
            g = r * _K + b
            w = _G0 if b % 2 == 0 else _G1
            wait_gather(b, g)

            @pl.when(g >= _K)
            def _():
                wait_out(b)

            @pl.loop(0, w, unroll=4)
            def _scale(row):
                for v in range(_D // _LANES):
                    sl = pl.ds(v * _LANES, _LANES)
                    sbuf[b, row, sl] = gbuf[b, row, sl] * _SCALE

            @pl.when(g + _K < n_groups)
            def _():
                start_gather(b, g + _K)

            start_out(b, g)

    for b in range(_K):
        wait_out(b)


@jax.jit
def kernel(x, table):
    batch, seq = x.shape
    rows_per_w = batch // _NW

    fn = pl.kernel(
        _body,
        out_type=jax.ShapeDtypeStruct((batch, seq, _D), jnp.float32),
        mesh=plsc.VectorSubcoreMesh(core_axis_name="c", subcore_axis_name="s"),
        scratch_types=[
            pltpu.VMEM((rows_per_w, seq), jnp.int32),
            pltpu.VMEM((_K, _G0, _D), jnp.float32),
            pltpu.VMEM((_K, _G0, _D), jnp.float32),
            pltpu.SemaphoreType.DMA((_K,)),
            pltpu.SemaphoreType.DMA((_K,)),
        ],
        compiler_params=pltpu.CompilerParams(use_tc_tiling_on_sc=False),
    )
    return fn(x.astype(jnp.int32), table)
